# Initial kernel scaffold; baseline (speedup 1.0000x reference)
import jax
import jax.numpy as jnp
from jax.experimental import pallas as pl


def kernel(X, emb_ty, vertex, edges, type, W, b, scale, eps):
    raise NotImplementedError("write your pallas kernel here")


# speedup vs baseline: 3.3034x; 3.3034x over previous
import functools
import math

import jax
import jax.numpy as jnp
from jax import lax
from jax.experimental import pallas as pl
from jax.experimental.pallas import tpu as pltpu
from jax.experimental.pallas import tpu_sc as plsc

N = 10000
E = 320000
NUM_HE = 20000
NUM_TY = 16
D = 128
H = D // 2

ROWS_BLK = 1000
CHUNK = 512
UNITS = CHUNK // 128
NCHUNK = E // CHUNK
NTILES = 16


def _lorentz_body(x_ref, w_ref, b_ref, s_ref, emb_ref, xl_ref, nemb_ref):
    x = x_ref[...]
    w = w_ref[...]
    h = lax.dot_general(x, w, (((1,), (1,)), ((), ())),
                        preferred_element_type=jnp.float32) + b_ref[...]
    sc = s_ref[0, 0]
    time = jax.nn.sigmoid(h[:, :1]) * jnp.exp(sc) + 1.1
    narrow = h[:, 1:]
    sq = jnp.clip(jnp.sum(narrow * narrow, axis=-1, keepdims=True), 1e-8, None)
    s = (time * time - 1.0) / sq
    full = jnp.concatenate([time, narrow * jnp.sqrt(s)], axis=1)
    xl_ref[0] = full[:, :H]
    xl_ref[1] = full[:, H:]

    @pl.when(pl.program_id(0) == 0)
    def _():
        e = emb_ref[...]
        nemb_ref[0] = -e[:, :H]
        nemb_ref[1] = -e[:, H:]


def _tc_lorentz(X, W, b2, scale2, emb_ty):
    grid = N // ROWS_BLK
    return pl.pallas_call(
        _lorentz_body,
        grid=(grid,),
        in_specs=[
            pl.BlockSpec((ROWS_BLK, D), lambda i: (i, 0)),
            pl.BlockSpec((D, D), lambda i: (0, 0)),
            pl.BlockSpec((1, D), lambda i: (0, 0)),
            pl.BlockSpec(memory_space=pltpu.SMEM),
            pl.BlockSpec((NUM_TY, D), lambda i: (0, 0)),
        ],
        out_specs=[
            pl.BlockSpec((2, ROWS_BLK, H), lambda i: (0, i, 0)),
            pl.BlockSpec((2, NUM_TY, H), lambda i: (0, 0, 0)),
        ],
        out_shape=[
            jax.ShapeDtypeStruct((2, N, H), jnp.float32),
            jax.ShapeDtypeStruct((2, NUM_TY, H), jnp.float32),
        ],
    )(X, W, b2, scale2, emb_ty)


def _sc_body(xl_hbm, nemb_hbm, vtx_hbm, edg_hbm, typ_hbm,
             xv_hbm, vidx, eidx, tidx, rows, xe_sh, xv_sh):
    c = lax.axis_index("c")
    s = lax.axis_index("s")

    z16 = jnp.zeros((16,), jnp.float32)

    def zrow(i, carry):
        for k in range(H // 16):
            rows[i, pl.ds(k * 16, 16)] = z16
        return carry

    lax.fori_loop(0, 125, zrow, 0)
    zs = NUM_HE // NTILES
    zv = N // NTILES
    for t in range(10):
        pltpu.sync_copy(rows.at[pl.ds(0, 125)],
                        xe_sh.at[pl.ds(s * zs + t * 125, 125)])
    for t in range(5):
        pltpu.sync_copy(rows.at[pl.ds(0, 125)],
                        xv_sh.at[pl.ds(s * zv + t * 125, 125)])
    plsc.subcore_barrier()

    voff = jnp.broadcast_to(c * N, (16,)).astype(jnp.int32)
    toff = jnp.broadcast_to(c * NUM_TY, (16,)).astype(jnp.int32)

    lo = (s * NCHUNK) // NTILES
    hi = ((s + 1) * NCHUNK) // NTILES

    def p1_body(ci, carry):
        base = ci * UNITS
        pltpu.sync_copy(vtx_hbm.at[pl.ds(base, UNITS)], vidx)
        pltpu.sync_copy(edg_hbm.at[pl.ds(base, UNITS)], eidx)
        pltpu.sync_copy(typ_hbm.at[pl.ds(base, UNITS)], tidx)
        for r in range(UNITS):
            for k in range(8):
                sl = pl.ds(k * 16, 16)
                vidx[r, sl] = vidx[r, sl] + voff
                tidx[r, sl] = tidx[r, sl] + toff
        for j in range(UNITS):
            pltpu.sync_copy(xl_hbm.at[vidx.at[j]], rows)
            pltpu.sync_copy(nemb_hbm.at[tidx.at[j]], rows, add=True)
            pltpu.sync_copy(rows, xe_sh.at[eidx.at[j]], add=True)
        return carry

    lax.fori_loop(lo, hi, p1_body, 0)
    plsc.subcore_barrier()

    def p2_body(ci, carry):
        base = ci * UNITS
        pltpu.sync_copy(vtx_hbm.at[pl.ds(base, UNITS)], vidx)
        pltpu.sync_copy(edg_hbm.at[pl.ds(base, UNITS)], eidx)
        for j in range(UNITS):
            pltpu.sync_copy(xe_sh.at[eidx.at[j]], rows)
            pltpu.sync_copy(rows, xv_sh.at[vidx.at[j]], add=True)
        return carry

    lax.fori_loop(lo, hi, p2_body, 0)
    plsc.subcore_barrier()

    pltpu.sync_copy(xv_sh.at[pl.ds(s * zv, zv)],
                    xv_hbm.at[c, pl.ds(s * zv, zv)])


_sc_prop = functools.partial(
    pl.kernel,
    out_type=jax.ShapeDtypeStruct((2, N, H), jnp.float32),
    mesh=plsc.VectorSubcoreMesh(core_axis_name="c", subcore_axis_name="s"),
    compiler_params=pltpu.CompilerParams(use_tc_tiling_on_sc=False),
    scratch_types=[
        pltpu.VMEM((UNITS, 128), jnp.int32),
        pltpu.VMEM((UNITS, 128), jnp.int32),
        pltpu.VMEM((UNITS, 128), jnp.int32),
        pltpu.VMEM((128, H), jnp.float32),
        pltpu.VMEM_SHARED((NUM_HE, H), jnp.float32),
        pltpu.VMEM_SHARED((N, H), jnp.float32),
    ],
)(_sc_body)


def _final_body(xl_ref, xv_ref, eps_ref, o_ref):
    e = eps_ref[0, 0]
    o_ref[:, :H] = e * xv_ref[0] + xl_ref[0]
    o_ref[:, H:] = e * xv_ref[1] + xl_ref[1]


def _tc_final(xl_split, xv_split, eps2):
    grid = N // ROWS_BLK
    return pl.pallas_call(
        _final_body,
        grid=(grid,),
        in_specs=[
            pl.BlockSpec((2, ROWS_BLK, H), lambda i: (0, i, 0)),
            pl.BlockSpec((2, ROWS_BLK, H), lambda i: (0, i, 0)),
            pl.BlockSpec(memory_space=pltpu.SMEM),
        ],
        out_specs=pl.BlockSpec((ROWS_BLK, D), lambda i: (i, 0)),
        out_shape=jax.ShapeDtypeStruct((N, D), jnp.float32),
    )(xl_split, xv_split, eps2)


def kernel(X, emb_ty, vertex, edges, type, W, b, scale, eps):
    b2 = b.reshape(1, D)
    scale2 = scale.reshape(1, 1)
    eps2 = eps.reshape(1, 1)
    xl_split, nemb_split = _tc_lorentz(X, W, b2, scale2, emb_ty)
    xl_tbl = xl_split.reshape(2 * N, H)
    nemb_tbl = nemb_split.reshape(2 * NUM_TY, H)
    vtx2 = vertex.reshape(E // 128, 128)
    edg2 = edges.reshape(E // 128, 128)
    typ2 = type.reshape(E // 128, 128)
    xv_split = _sc_prop(xl_tbl, nemb_tbl, vtx2, edg2, typ2)
    return _tc_final(xl_split, xv_split, eps2)


# speedup vs baseline: 3.5721x; 1.0813x over previous
import functools

import jax
import jax.numpy as jnp
from jax import lax
from jax.experimental import pallas as pl
from jax.experimental.pallas import tpu as pltpu
from jax.experimental.pallas import tpu_sc as plsc

N = 10000
E = 320000
NUM_HE = 20000
NUM_TY = 16
D = 128
H = D // 2

ROWS_BLK = 1000
GROUP = 512
NB = 4
NGROUP = E // GROUP
NTILES = 16


def _lorentz_body(x_ref, w_ref, b_ref, s_ref, emb_ref, xl_ref, nemb_ref):
    x = x_ref[...]
    w = w_ref[...]
    h = lax.dot_general(x, w, (((1,), (1,)), ((), ())),
                        preferred_element_type=jnp.float32) + b_ref[...]
    sc = s_ref[0, 0]
    time = jax.nn.sigmoid(h[:, :1]) * jnp.exp(sc) + 1.1
    narrow = h[:, 1:]
    sq = jnp.clip(jnp.sum(narrow * narrow, axis=-1, keepdims=True), 1e-8, None)
    s = (time * time - 1.0) / sq
    full = jnp.concatenate([time, narrow * jnp.sqrt(s)], axis=1)
    xl_ref[0] = full[:, :H]
    xl_ref[1] = full[:, H:]

    @pl.when(pl.program_id(0) == 0)
    def _():
        e = emb_ref[...]
        nemb_ref[0] = -e[:, :H]
        nemb_ref[1] = -e[:, H:]


def _tc_lorentz(X, W, b2, scale2, emb_ty):
    grid = N // ROWS_BLK
    return pl.pallas_call(
        _lorentz_body,
        grid=(grid,),
        in_specs=[
            pl.BlockSpec((ROWS_BLK, D), lambda i: (i, 0)),
            pl.BlockSpec((D, D), lambda i: (0, 0)),
            pl.BlockSpec((1, D), lambda i: (0, 0)),
            pl.BlockSpec(memory_space=pltpu.SMEM),
            pl.BlockSpec((NUM_TY, D), lambda i: (0, 0)),
        ],
        out_specs=[
            pl.BlockSpec((2, ROWS_BLK, H), lambda i: (0, i, 0)),
            pl.BlockSpec((2, NUM_TY, H), lambda i: (0, 0, 0)),
        ],
        out_shape=[
            jax.ShapeDtypeStruct((2, N, H), jnp.float32),
            jax.ShapeDtypeStruct((2, NUM_TY, H), jnp.float32),
        ],
    )(X, W, b2, scale2, emb_ty)


def _zero_shared(rows0, sh, base, nrows):
    z16 = jnp.zeros((16,), jnp.float32)

    def zrow(i, carry):
        for k in range(H // 16):
            rows0[i, pl.ds(k * 16, 16)] = z16
        return carry

    lax.fori_loop(0, 125, zrow, 0)
    for t in range(nrows // 125):
        pltpu.sync_copy(rows0.at[pl.ds(0, 125)],
                        sh.at[pl.ds(base + t * 125, 125)])


def _sc_edges_body(xl_hbm, nemb_hbm, vtx_hbm, edg_hbm, typ_hbm,
                   xe_hbm, vA, eA, tA, vB, eB, tB, r0, r1, r2, r3,
                   isemA, isemB, gsem, asem, ssem, xe_sh):
    c = lax.axis_index("c")
    s = lax.axis_index("s")
    rows = [r0, r1, r2, r3]

    zs = NUM_HE // NTILES
    _zero_shared(r0, xe_sh, s * zs, zs)
    plsc.subcore_barrier()

    voff = jnp.broadcast_to(c * N, (16,)).astype(jnp.int32)
    toff = jnp.broadcast_to(c * NUM_TY, (16,)).astype(jnp.int32)
    lo = (s * NGROUP) // NTILES
    hi = ((s + 1) * NGROUP) // NTILES

    def load_idx(g, idx, sem):
        v, e, t = idx
        base = g * NB
        return [pltpu.async_copy(vtx_hbm.at[pl.ds(base, NB)], v, sem),
                pltpu.async_copy(edg_hbm.at[pl.ds(base, NB)], e, sem),
                pltpu.async_copy(typ_hbm.at[pl.ds(base, NB)], t, sem)]

    A = (vA, eA, tA)
    B = (vB, eB, tB)

    def adjust(idx):
        v, e, t = idx
        for r in range(NB):
            for k in range(8):
                sl = pl.ds(k * 16, 16)
                v[r, sl] = v[r, sl] + voff
                t[r, sl] = t[r, sl] + toff

    def legs(idx):
        v, e, t = idx
        gd = [pltpu.async_copy(xl_hbm.at[v.at[b]], rows[b], gsem)
              for b in range(NB)]
        for d in gd:
            d.wait()
        ad = [pltpu.async_copy(nemb_hbm.at[t.at[b]], rows[b], asem,
                               add=True) for b in range(NB)]
        for d in ad:
            d.wait()
        sd = [pltpu.async_copy(rows[b], xe_sh.at[e.at[b]], ssem, add=True)
              for b in range(NB)]
        for d in sd:
            d.wait()

    def emit_pair(g0, g1):
        dA = load_idx(g0, A, isemA)
        dB = load_idx(g1, B, isemB)
        for d in dA:
            d.wait()
        adjust(A)
        legs(A)
        for d in dB:
            d.wait()
        adjust(B)
        legs(B)

    def pair_body(k, carry):
        g0 = lo + 2 * k
        emit_pair(g0, g0 + 1)
        return carry

    lax.fori_loop(0, (hi - lo) // 2, pair_body, 0)

    @pl.when(lax.rem(hi - lo, 2) == 1)
    def _():
        dA = load_idx(hi - 1, A, isemA)
        for d in dA:
            d.wait()
        adjust(A)
        legs(A)

    plsc.subcore_barrier()

    pltpu.sync_copy(xe_sh.at[pl.ds(s * zs, zs)],
                    xe_hbm.at[c, pl.ds(s * zs, zs)])


_sc_edges = functools.partial(
    pl.kernel,
    out_type=jax.ShapeDtypeStruct((2, NUM_HE, H), jnp.float32),
    mesh=plsc.VectorSubcoreMesh(core_axis_name="c", subcore_axis_name="s"),
    compiler_params=pltpu.CompilerParams(use_tc_tiling_on_sc=False),
    scratch_types=[
        pltpu.VMEM((NB, 128), jnp.int32),
        pltpu.VMEM((NB, 128), jnp.int32),
        pltpu.VMEM((NB, 128), jnp.int32),
        pltpu.VMEM((NB, 128), jnp.int32),
        pltpu.VMEM((NB, 128), jnp.int32),
        pltpu.VMEM((NB, 128), jnp.int32),
        pltpu.VMEM((128, H), jnp.float32),
        pltpu.VMEM((128, H), jnp.float32),
        pltpu.VMEM((128, H), jnp.float32),
        pltpu.VMEM((128, H), jnp.float32),
        pltpu.SemaphoreType.DMA,
        pltpu.SemaphoreType.DMA,
        pltpu.SemaphoreType.DMA,
        pltpu.SemaphoreType.DMA,
        pltpu.SemaphoreType.DMA,
        pltpu.VMEM_SHARED((NUM_HE, H), jnp.float32),
    ],
)(_sc_edges_body)


def _sc_verts_body(xe_hbm, vtx_hbm, edg_hbm,
                   xv_hbm, vA, eA, vB, eB, r0, r1, r2, r3,
                   isemA, isemB, gsem, ssem, xv_sh):
    c = lax.axis_index("c")
    s = lax.axis_index("s")
    rows = [r0, r1, r2, r3]

    zv = N // NTILES
    _zero_shared(r0, xv_sh, s * zv, zv)
    plsc.subcore_barrier()

    eoff = jnp.broadcast_to(c * NUM_HE, (16,)).astype(jnp.int32)
    lo = (s * NGROUP) // NTILES
    hi = ((s + 1) * NGROUP) // NTILES

    def load_idx(g, idx, sem):
        v, e = idx
        base = g * NB
        return [pltpu.async_copy(vtx_hbm.at[pl.ds(base, NB)], v, sem),
                pltpu.async_copy(edg_hbm.at[pl.ds(base, NB)], e, sem)]

    A = (vA, eA)
    B = (vB, eB)

    def adjust(idx):
        v, e = idx
        for r in range(NB):
            for k in range(8):
                sl = pl.ds(k * 16, 16)
                e[r, sl] = e[r, sl] + eoff

    def legs(idx):
        v, e = idx
        gd = [pltpu.async_copy(xe_hbm.at[e.at[b]], rows[b], gsem)
              for b in range(NB)]
        for d in gd:
            d.wait()
        sd = [pltpu.async_copy(rows[b], xv_sh.at[v.at[b]], ssem, add=True)
              for b in range(NB)]
        for d in sd:
            d.wait()

    def emit_pair(g0, g1):
        dA = load_idx(g0, A, isemA)
        dB = load_idx(g1, B, isemB)
        for d in dA:
            d.wait()
        adjust(A)
        legs(A)
        for d in dB:
            d.wait()
        adjust(B)
        legs(B)

    def pair_body(k, carry):
        g0 = lo + 2 * k
        emit_pair(g0, g0 + 1)
        return carry

    lax.fori_loop(0, (hi - lo) // 2, pair_body, 0)

    @pl.when(lax.rem(hi - lo, 2) == 1)
    def _():
        dA = load_idx(hi - 1, A, isemA)
        for d in dA:
            d.wait()
        adjust(A)
        legs(A)

    plsc.subcore_barrier()

    pltpu.sync_copy(xv_sh.at[pl.ds(s * zv, zv)],
                    xv_hbm.at[c, pl.ds(s * zv, zv)])


_sc_verts = functools.partial(
    pl.kernel,
    out_type=jax.ShapeDtypeStruct((2, N, H), jnp.float32),
    mesh=plsc.VectorSubcoreMesh(core_axis_name="c", subcore_axis_name="s"),
    compiler_params=pltpu.CompilerParams(use_tc_tiling_on_sc=False),
    scratch_types=[
        pltpu.VMEM((NB, 128), jnp.int32),
        pltpu.VMEM((NB, 128), jnp.int32),
        pltpu.VMEM((NB, 128), jnp.int32),
        pltpu.VMEM((NB, 128), jnp.int32),
        pltpu.VMEM((128, H), jnp.float32),
        pltpu.VMEM((128, H), jnp.float32),
        pltpu.VMEM((128, H), jnp.float32),
        pltpu.VMEM((128, H), jnp.float32),
        pltpu.SemaphoreType.DMA,
        pltpu.SemaphoreType.DMA,
        pltpu.SemaphoreType.DMA,
        pltpu.SemaphoreType.DMA,
        pltpu.VMEM_SHARED((N, H), jnp.float32),
    ],
)(_sc_verts_body)


def _final_body(xl_ref, xv_ref, eps_ref, o_ref):
    e = eps_ref[0, 0]
    o_ref[:, :H] = e * xv_ref[0] + xl_ref[0]
    o_ref[:, H:] = e * xv_ref[1] + xl_ref[1]


def _tc_final(xl_split, xv_split, eps2):
    grid = N // ROWS_BLK
    return pl.pallas_call(
        _final_body,
        grid=(grid,),
        in_specs=[
            pl.BlockSpec((2, ROWS_BLK, H), lambda i: (0, i, 0)),
            pl.BlockSpec((2, ROWS_BLK, H), lambda i: (0, i, 0)),
            pl.BlockSpec(memory_space=pltpu.SMEM),
        ],
        out_specs=pl.BlockSpec((ROWS_BLK, D), lambda i: (i, 0)),
        out_shape=jax.ShapeDtypeStruct((N, D), jnp.float32),
    )(xl_split, xv_split, eps2)


def kernel(X, emb_ty, vertex, edges, type, W, b, scale, eps):
    b2 = b.reshape(1, D)
    scale2 = scale.reshape(1, 1)
    eps2 = eps.reshape(1, 1)
    xl_split, nemb_split = _tc_lorentz(X, W, b2, scale2, emb_ty)
    xl_tbl = xl_split.reshape(2 * N, H)
    nemb_tbl = nemb_split.reshape(2 * NUM_TY, H)
    vtx2 = vertex.reshape(E // 128, 128)
    edg2 = edges.reshape(E // 128, 128)
    typ2 = type.reshape(E // 128, 128)
    xe_split = _sc_edges(xl_tbl, nemb_tbl, vtx2, edg2, typ2)
    xe_tbl = xe_split.reshape(2 * NUM_HE, H)
    xv_split = _sc_verts(xe_tbl, vtx2, edg2)
    return _tc_final(xl_split, xv_split, eps2)


# speedup vs baseline: 7.4419x; 2.0833x over previous
import functools

import jax
import jax.numpy as jnp
from jax import lax
from jax.experimental import pallas as pl
from jax.experimental.pallas import tpu as pltpu
from jax.experimental.pallas import tpu_sc as plsc

N = 10000
E = 320000
NUM_HE = 20000
NUM_TY = 16
D = 128
H = D // 2

ROWS_BLK = 1000
GROUP = 512
NB = 4
NGROUP = E // GROUP
NTILES = 16


def _lorentz_body(x_ref, w_ref, b_ref, s_ref, emb_ref, xl_ref, xlvt_ref):
    x = x_ref[...]
    w = w_ref[...]
    h = lax.dot_general(x, w, (((1,), (1,)), ((), ())),
                        preferred_element_type=jnp.float32) + b_ref[...]
    sc = s_ref[0, 0]
    time = jax.nn.sigmoid(h[:, :1]) * jnp.exp(sc) + 1.1
    narrow = h[:, 1:]
    sq = jnp.clip(jnp.sum(narrow * narrow, axis=-1, keepdims=True), 1e-8, None)
    s = (time * time - 1.0) / sq
    full = jnp.concatenate([time, narrow * jnp.sqrt(s)], axis=1)
    e = emb_ref[...]
    xl_ref[0] = full[:, :H]
    xl_ref[1] = full[:, H:]
    xlvt_ref[0] = full[:, :H][:, None, :] - e[:, :H][None, :, :]
    xlvt_ref[1] = full[:, H:][:, None, :] - e[:, H:][None, :, :]


def _tc_lorentz(X, W, b2, scale2, emb_ty):
    grid = N // ROWS_BLK
    return pl.pallas_call(
        _lorentz_body,
        grid=(grid,),
        in_specs=[
            pl.BlockSpec((ROWS_BLK, D), lambda i: (i, 0)),
            pl.BlockSpec((D, D), lambda i: (0, 0)),
            pl.BlockSpec((1, D), lambda i: (0, 0)),
            pl.BlockSpec(memory_space=pltpu.SMEM),
            pl.BlockSpec((NUM_TY, D), lambda i: (0, 0)),
        ],
        out_specs=[
            pl.BlockSpec((2, ROWS_BLK, H), lambda i: (0, i, 0)),
            pl.BlockSpec((2, ROWS_BLK, NUM_TY, H), lambda i: (0, i, 0, 0)),
        ],
        out_shape=[
            jax.ShapeDtypeStruct((2, N, H), jnp.float32),
            jax.ShapeDtypeStruct((2, N, NUM_TY, H), jnp.float32),
        ],
    )(X, W, b2, scale2, emb_ty)


def _zero_shared(rows0, sh, base, nrows):
    z16 = jnp.zeros((16,), jnp.float32)

    def zrow(i, carry):
        for k in range(H // 16):
            rows0[i, pl.ds(k * 16, 16)] = z16
        return carry

    lax.fori_loop(0, 125, zrow, 0)
    for t in range(nrows // 125):
        pltpu.sync_copy(rows0.at[pl.ds(0, 125)],
                        sh.at[pl.ds(base + t * 125, 125)])


def _sc_edges_body(xlvt_hbm, vtx_hbm, edg_hbm, typ_hbm,
                   xe_hbm, vA, eA, tA, vB, eB, tB, r0, r1, r2, r3,
                   isemA, isemB, gsem, ssem, xe_sh):
    c = lax.axis_index("c")
    s = lax.axis_index("s")
    rows = [r0, r1, r2, r3]

    zs = NUM_HE // NTILES
    _zero_shared(r0, xe_sh, s * zs, zs)
    plsc.subcore_barrier()

    coff = jnp.broadcast_to(c * (N * NUM_TY), (16,)).astype(jnp.int32)
    lo = (s * NGROUP) // NTILES
    hi = ((s + 1) * NGROUP) // NTILES

    def load_idx(g, idx, sem):
        v, e, t = idx
        base = g * NB
        return [pltpu.async_copy(vtx_hbm.at[pl.ds(base, NB)], v, sem),
                pltpu.async_copy(edg_hbm.at[pl.ds(base, NB)], e, sem),
                pltpu.async_copy(typ_hbm.at[pl.ds(base, NB)], t, sem)]

    A = (vA, eA, tA)
    B = (vB, eB, tB)

    def adjust(idx):
        v, e, t = idx
        for r in range(NB):
            for k in range(8):
                sl = pl.ds(k * 16, 16)
                v[r, sl] = v[r, sl] * NUM_TY + t[r, sl] + coff

    def legs(idx):
        v, e, t = idx
        gd = [pltpu.async_copy(xlvt_hbm.at[v.at[b]], rows[b], gsem)
              for b in range(NB)]
        for d in gd:
            d.wait()
        sd = [pltpu.async_copy(rows[b], xe_sh.at[e.at[b]], ssem, add=True)
              for b in range(NB)]
        for d in sd:
            d.wait()

    def emit_pair(g0, g1):
        dA = load_idx(g0, A, isemA)
        dB = load_idx(g1, B, isemB)
        for d in dA:
            d.wait()
        adjust(A)
        legs(A)
        for d in dB:
            d.wait()
        adjust(B)
        legs(B)

    def pair_body(k, carry):
        g0 = lo + 2 * k
        emit_pair(g0, g0 + 1)
        return carry

    lax.fori_loop(0, (hi - lo) // 2, pair_body, 0)

    @pl.when(lax.rem(hi - lo, 2) == 1)
    def _():
        dA = load_idx(hi - 1, A, isemA)
        for d in dA:
            d.wait()
        adjust(A)
        legs(A)

    plsc.subcore_barrier()

    pltpu.sync_copy(xe_sh.at[pl.ds(s * zs, zs)],
                    xe_hbm.at[c, pl.ds(s * zs, zs)])


_sc_edges = functools.partial(
    pl.kernel,
    out_type=jax.ShapeDtypeStruct((2, NUM_HE, H), jnp.float32),
    mesh=plsc.VectorSubcoreMesh(core_axis_name="c", subcore_axis_name="s"),
    compiler_params=pltpu.CompilerParams(use_tc_tiling_on_sc=False),
    scratch_types=[
        pltpu.VMEM((NB, 128), jnp.int32),
        pltpu.VMEM((NB, 128), jnp.int32),
        pltpu.VMEM((NB, 128), jnp.int32),
        pltpu.VMEM((NB, 128), jnp.int32),
        pltpu.VMEM((NB, 128), jnp.int32),
        pltpu.VMEM((NB, 128), jnp.int32),
        pltpu.VMEM((128, H), jnp.float32),
        pltpu.VMEM((128, H), jnp.float32),
        pltpu.VMEM((128, H), jnp.float32),
        pltpu.VMEM((128, H), jnp.float32),
        pltpu.SemaphoreType.DMA,
        pltpu.SemaphoreType.DMA,
        pltpu.SemaphoreType.DMA,
        pltpu.SemaphoreType.DMA,
        pltpu.VMEM_SHARED((NUM_HE, H), jnp.float32),
    ],
)(_sc_edges_body)


def _sc_verts_body(xe_hbm, vtx_hbm, edg_hbm,
                   xv_hbm, vA, eA, vB, eB, r0, r1, r2, r3,
                   isemA, isemB, gsem, ssem, xv_sh):
    c = lax.axis_index("c")
    s = lax.axis_index("s")
    rows = [r0, r1, r2, r3]

    zv = N // NTILES
    _zero_shared(r0, xv_sh, s * zv, zv)
    plsc.subcore_barrier()

    eoff = jnp.broadcast_to(c * NUM_HE, (16,)).astype(jnp.int32)
    lo = (s * NGROUP) // NTILES
    hi = ((s + 1) * NGROUP) // NTILES

    def load_idx(g, idx, sem):
        v, e = idx
        base = g * NB
        return [pltpu.async_copy(vtx_hbm.at[pl.ds(base, NB)], v, sem),
                pltpu.async_copy(edg_hbm.at[pl.ds(base, NB)], e, sem)]

    A = (vA, eA)
    B = (vB, eB)

    def adjust(idx):
        v, e = idx
        for r in range(NB):
            for k in range(8):
                sl = pl.ds(k * 16, 16)
                e[r, sl] = e[r, sl] + eoff

    def legs(idx):
        v, e = idx
        gd = [pltpu.async_copy(xe_hbm.at[e.at[b]], rows[b], gsem)
              for b in range(NB)]
        for d in gd:
            d.wait()
        sd = [pltpu.async_copy(rows[b], xv_sh.at[v.at[b]], ssem, add=True)
              for b in range(NB)]
        for d in sd:
            d.wait()

    def emit_pair(g0, g1):
        dA = load_idx(g0, A, isemA)
        dB = load_idx(g1, B, isemB)
        for d in dA:
            d.wait()
        adjust(A)
        legs(A)
        for d in dB:
            d.wait()
        adjust(B)
        legs(B)

    def pair_body(k, carry):
        g0 = lo + 2 * k
        emit_pair(g0, g0 + 1)
        return carry

    lax.fori_loop(0, (hi - lo) // 2, pair_body, 0)

    @pl.when(lax.rem(hi - lo, 2) == 1)
    def _():
        dA = load_idx(hi - 1, A, isemA)
        for d in dA:
            d.wait()
        adjust(A)
        legs(A)

    plsc.subcore_barrier()

    pltpu.sync_copy(xv_sh.at[pl.ds(s * zv, zv)],
                    xv_hbm.at[c, pl.ds(s * zv, zv)])


_sc_verts = functools.partial(
    pl.kernel,
    out_type=jax.ShapeDtypeStruct((2, N, H), jnp.float32),
    mesh=plsc.VectorSubcoreMesh(core_axis_name="c", subcore_axis_name="s"),
    compiler_params=pltpu.CompilerParams(use_tc_tiling_on_sc=False),
    scratch_types=[
        pltpu.VMEM((NB, 128), jnp.int32),
        pltpu.VMEM((NB, 128), jnp.int32),
        pltpu.VMEM((NB, 128), jnp.int32),
        pltpu.VMEM((NB, 128), jnp.int32),
        pltpu.VMEM((128, H), jnp.float32),
        pltpu.VMEM((128, H), jnp.float32),
        pltpu.VMEM((128, H), jnp.float32),
        pltpu.VMEM((128, H), jnp.float32),
        pltpu.SemaphoreType.DMA,
        pltpu.SemaphoreType.DMA,
        pltpu.SemaphoreType.DMA,
        pltpu.SemaphoreType.DMA,
        pltpu.VMEM_SHARED((N, H), jnp.float32),
    ],
)(_sc_verts_body)


def _final_body(xl_ref, xv_ref, eps_ref, o_ref):
    e = eps_ref[0, 0]
    o_ref[:, :H] = e * xv_ref[0] + xl_ref[0]
    o_ref[:, H:] = e * xv_ref[1] + xl_ref[1]


def _tc_final(xl_split, xv_split, eps2):
    grid = N // ROWS_BLK
    return pl.pallas_call(
        _final_body,
        grid=(grid,),
        in_specs=[
            pl.BlockSpec((2, ROWS_BLK, H), lambda i: (0, i, 0)),
            pl.BlockSpec((2, ROWS_BLK, H), lambda i: (0, i, 0)),
            pl.BlockSpec(memory_space=pltpu.SMEM),
        ],
        out_specs=pl.BlockSpec((ROWS_BLK, D), lambda i: (i, 0)),
        out_shape=jax.ShapeDtypeStruct((N, D), jnp.float32),
    )(xl_split, xv_split, eps2)


def kernel(X, emb_ty, vertex, edges, type, W, b, scale, eps):
    b2 = b.reshape(1, D)
    scale2 = scale.reshape(1, 1)
    eps2 = eps.reshape(1, 1)
    xl_split, xlvt_split = _tc_lorentz(X, W, b2, scale2, emb_ty)
    xlvt_tbl = xlvt_split.reshape(2 * N * NUM_TY, H)
    vtx2 = vertex.reshape(E // 128, 128)
    edg2 = edges.reshape(E // 128, 128)
    typ2 = type.reshape(E // 128, 128)
    xe_split = _sc_edges(xlvt_tbl, vtx2, edg2, typ2)
    xe_tbl = xe_split.reshape(2 * NUM_HE, H)
    xv_split = _sc_verts(xe_tbl, vtx2, edg2)
    return _tc_final(xl_split, xv_split, eps2)


# speedup vs baseline: 7.6703x; 1.0307x over previous
import functools

import jax
import jax.numpy as jnp
from jax import lax
from jax.experimental import pallas as pl
from jax.experimental.pallas import tpu as pltpu
from jax.experimental.pallas import tpu_sc as plsc

N = 10000
E = 320000
NUM_HE = 20000
NUM_TY = 16
D = 128
H = D // 2

ROWS_BLK = 1000
GROUP = 512
NB = 4
NGROUP = E // GROUP
NTILES = 16


def _lorentz_body(x_ref, w_ref, b_ref, s_ref, emb_ref, xl_ref, xlvt_ref):
    x = x_ref[...]
    w = w_ref[...]
    h = lax.dot_general(x, w, (((1,), (1,)), ((), ())),
                        preferred_element_type=jnp.float32) + b_ref[...]
    sc = s_ref[0, 0]
    time = jax.nn.sigmoid(h[:, :1]) * jnp.exp(sc) + 1.1
    narrow = h[:, 1:]
    sq = jnp.clip(jnp.sum(narrow * narrow, axis=-1, keepdims=True), 1e-8, None)
    s = (time * time - 1.0) / sq
    full = jnp.concatenate([time, narrow * jnp.sqrt(s)], axis=1)
    e = emb_ref[...]
    xl_ref[0] = full[:, :H]
    xl_ref[1] = full[:, H:]
    xlvt_ref[0] = full[:, :H][:, None, :] - e[:, :H][None, :, :]
    xlvt_ref[1] = full[:, H:][:, None, :] - e[:, H:][None, :, :]


def _tc_lorentz(X, W, b2, scale2, emb_ty):
    grid = N // ROWS_BLK
    return pl.pallas_call(
        _lorentz_body,
        grid=(grid,),
        in_specs=[
            pl.BlockSpec((ROWS_BLK, D), lambda i: (i, 0)),
            pl.BlockSpec((D, D), lambda i: (0, 0)),
            pl.BlockSpec((1, D), lambda i: (0, 0)),
            pl.BlockSpec(memory_space=pltpu.SMEM),
            pl.BlockSpec((NUM_TY, D), lambda i: (0, 0)),
        ],
        out_specs=[
            pl.BlockSpec((2, ROWS_BLK, H), lambda i: (0, i, 0)),
            pl.BlockSpec((2, ROWS_BLK, NUM_TY, H), lambda i: (0, i, 0, 0)),
        ],
        out_shape=[
            jax.ShapeDtypeStruct((2, N, H), jnp.float32),
            jax.ShapeDtypeStruct((2, N, NUM_TY, H), jnp.float32),
        ],
    )(X, W, b2, scale2, emb_ty)


def _zero_shared(rows0, sh, base, nrows):
    z16 = jnp.zeros((16,), jnp.float32)

    def zrow(i, carry):
        for k in range(H // 16):
            rows0[i, pl.ds(k * 16, 16)] = z16
        return carry

    lax.fori_loop(0, 125, zrow, 0)
    for t in range(nrows // 125):
        pltpu.sync_copy(rows0.at[pl.ds(0, 125)],
                        sh.at[pl.ds(base + t * 125, 125)])


def _sc_edges_body(xlvt_hbm, vtx_hbm, edg_hbm, typ_hbm,
                   xe_hbm, vA, eA, tA, vB, eB, tB, r0, r1, r2, r3,
                   isemA, isemB, gsem, ssem, xe_sh):
    c = lax.axis_index("c")
    s = lax.axis_index("s")
    rows = [r0, r1, r2, r3]

    zs = NUM_HE // NTILES
    _zero_shared(r0, xe_sh, s * zs, zs)
    plsc.subcore_barrier()

    coff = jnp.broadcast_to(c * (N * NUM_TY), (16,)).astype(jnp.int32)
    lo = (s * NGROUP) // NTILES
    hi = ((s + 1) * NGROUP) // NTILES

    def load_idx(g, idx, sem):
        v, e, t = idx
        base = g * NB
        return [pltpu.async_copy(vtx_hbm.at[pl.ds(base, NB)], v, sem),
                pltpu.async_copy(edg_hbm.at[pl.ds(base, NB)], e, sem),
                pltpu.async_copy(typ_hbm.at[pl.ds(base, NB)], t, sem)]

    A = (vA, eA, tA)
    B = (vB, eB, tB)

    def adjust(idx):
        v, e, t = idx
        for r in range(NB):
            for k in range(8):
                sl = pl.ds(k * 16, 16)
                v[r, sl] = v[r, sl] * NUM_TY + t[r, sl] + coff

    def legs(idx):
        v, e, t = idx
        gd = [pltpu.async_copy(xlvt_hbm.at[v.at[b]], rows[b], gsem)
              for b in range(NB)]
        for d in gd:
            d.wait()
        sd = [pltpu.async_copy(rows[b], xe_sh.at[e.at[b]], ssem, add=True)
              for b in range(NB)]
        for d in sd:
            d.wait()

    def emit_pair(g0, g1):
        dA = load_idx(g0, A, isemA)
        dB = load_idx(g1, B, isemB)
        for d in dA:
            d.wait()
        adjust(A)
        legs(A)
        for d in dB:
            d.wait()
        adjust(B)
        legs(B)

    def pair_body(k, carry):
        g0 = lo + 2 * k
        emit_pair(g0, g0 + 1)
        return carry

    lax.fori_loop(0, (hi - lo) // 2, pair_body, 0)

    @pl.when(lax.rem(hi - lo, 2) == 1)
    def _():
        dA = load_idx(hi - 1, A, isemA)
        for d in dA:
            d.wait()
        adjust(A)
        legs(A)

    plsc.subcore_barrier()

    pltpu.sync_copy(xe_sh.at[pl.ds(s * zs, zs)],
                    xe_hbm.at[c, pl.ds(s * zs, zs)])


_sc_edges = functools.partial(
    pl.kernel,
    out_type=jax.ShapeDtypeStruct((2, NUM_HE, H), jnp.float32),
    mesh=plsc.VectorSubcoreMesh(core_axis_name="c", subcore_axis_name="s"),
    compiler_params=pltpu.CompilerParams(use_tc_tiling_on_sc=False),
    scratch_types=[
        pltpu.VMEM((NB, 128), jnp.int32),
        pltpu.VMEM((NB, 128), jnp.int32),
        pltpu.VMEM((NB, 128), jnp.int32),
        pltpu.VMEM((NB, 128), jnp.int32),
        pltpu.VMEM((NB, 128), jnp.int32),
        pltpu.VMEM((NB, 128), jnp.int32),
        pltpu.VMEM((128, H), jnp.float32),
        pltpu.VMEM((128, H), jnp.float32),
        pltpu.VMEM((128, H), jnp.float32),
        pltpu.VMEM((128, H), jnp.float32),
        pltpu.SemaphoreType.DMA,
        pltpu.SemaphoreType.DMA,
        pltpu.SemaphoreType.DMA,
        pltpu.SemaphoreType.DMA,
        pltpu.VMEM_SHARED((NUM_HE, H), jnp.float32),
    ],
)(_sc_edges_body)


def _sc_verts_body(xe_hbm, vtx_hbm, edg_hbm,
                   xv_hbm, vA, eA, vB, eB, r0, r1, r2, r3,
                   r4, r5, r6, r7,
                   isemA, isemB, gsem, ssem, xv_sh):
    c = lax.axis_index("c")
    s = lax.axis_index("s")
    bank0 = [r0, r1, r2, r3]
    bank1 = [r4, r5, r6, r7]

    zv = N // NTILES
    _zero_shared(r0, xv_sh, s * zv, zv)
    plsc.subcore_barrier()

    eoff = jnp.broadcast_to(c * NUM_HE, (16,)).astype(jnp.int32)
    lo = (s * NGROUP) // NTILES
    hi = ((s + 1) * NGROUP) // NTILES

    def load_idx(g, idx, sem):
        v, e = idx
        base = g * NB
        return [pltpu.async_copy(vtx_hbm.at[pl.ds(base, NB)], v, sem),
                pltpu.async_copy(edg_hbm.at[pl.ds(base, NB)], e, sem)]

    A = (vA, eA)
    B = (vB, eB)

    def adjust(idx):
        v, e = idx
        for r in range(NB):
            for k in range(8):
                sl = pl.ds(k * 16, 16)
                e[r, sl] = e[r, sl] + eoff

    def gathers(idx, rows):
        v, e = idx
        gd = [pltpu.async_copy(xe_hbm.at[e.at[b]], rows[b], gsem)
              for b in range(NB)]
        for d in gd:
            d.wait()

    def scatters(idx, rows):
        v, e = idx
        return [pltpu.async_copy(rows[b], xv_sh.at[v.at[b]], ssem, add=True)
                for b in range(NB)]

    def emit_pair(g0, g1):
        dA = load_idx(g0, A, isemA)
        dB = load_idx(g1, B, isemB)
        for d in dA:
            d.wait()
        adjust(A)
        gathers(A, bank0)
        sd0 = scatters(A, bank0)
        for d in dB:
            d.wait()
        adjust(B)
        gathers(B, bank1)
        for d in sd0:
            d.wait()
        sd1 = scatters(B, bank1)
        for d in sd1:
            d.wait()

    def pair_body(k, carry):
        g0 = lo + 2 * k
        emit_pair(g0, g0 + 1)
        return carry

    lax.fori_loop(0, (hi - lo) // 2, pair_body, 0)

    @pl.when(lax.rem(hi - lo, 2) == 1)
    def _():
        dA = load_idx(hi - 1, A, isemA)
        for d in dA:
            d.wait()
        adjust(A)
        gathers(A, bank0)
        for d in scatters(A, bank0):
            d.wait()

    plsc.subcore_barrier()

    pltpu.sync_copy(xv_sh.at[pl.ds(s * zv, zv)],
                    xv_hbm.at[c, pl.ds(s * zv, zv)])


_sc_verts = functools.partial(
    pl.kernel,
    out_type=jax.ShapeDtypeStruct((2, N, H), jnp.float32),
    mesh=plsc.VectorSubcoreMesh(core_axis_name="c", subcore_axis_name="s"),
    compiler_params=pltpu.CompilerParams(use_tc_tiling_on_sc=False),
    scratch_types=[
        pltpu.VMEM((NB, 128), jnp.int32),
        pltpu.VMEM((NB, 128), jnp.int32),
        pltpu.VMEM((NB, 128), jnp.int32),
        pltpu.VMEM((NB, 128), jnp.int32),
        pltpu.VMEM((128, H), jnp.float32),
        pltpu.VMEM((128, H), jnp.float32),
        pltpu.VMEM((128, H), jnp.float32),
        pltpu.VMEM((128, H), jnp.float32),
        pltpu.VMEM((128, H), jnp.float32),
        pltpu.VMEM((128, H), jnp.float32),
        pltpu.VMEM((128, H), jnp.float32),
        pltpu.VMEM((128, H), jnp.float32),
        pltpu.SemaphoreType.DMA,
        pltpu.SemaphoreType.DMA,
        pltpu.SemaphoreType.DMA,
        pltpu.SemaphoreType.DMA,
        pltpu.VMEM_SHARED((N, H), jnp.float32),
    ],
)(_sc_verts_body)


def _final_body(xl_ref, xv_ref, eps_ref, o_ref):
    e = eps_ref[0, 0]
    o_ref[:, :H] = e * xv_ref[0] + xl_ref[0]
    o_ref[:, H:] = e * xv_ref[1] + xl_ref[1]


def _tc_final(xl_split, xv_split, eps2):
    grid = N // ROWS_BLK
    return pl.pallas_call(
        _final_body,
        grid=(grid,),
        in_specs=[
            pl.BlockSpec((2, ROWS_BLK, H), lambda i: (0, i, 0)),
            pl.BlockSpec((2, ROWS_BLK, H), lambda i: (0, i, 0)),
            pl.BlockSpec(memory_space=pltpu.SMEM),
        ],
        out_specs=pl.BlockSpec((ROWS_BLK, D), lambda i: (i, 0)),
        out_shape=jax.ShapeDtypeStruct((N, D), jnp.float32),
    )(xl_split, xv_split, eps2)


def kernel(X, emb_ty, vertex, edges, type, W, b, scale, eps):
    b2 = b.reshape(1, D)
    scale2 = scale.reshape(1, 1)
    eps2 = eps.reshape(1, 1)
    xl_split, xlvt_split = _tc_lorentz(X, W, b2, scale2, emb_ty)
    xlvt_tbl = xlvt_split.reshape(2 * N * NUM_TY, H)
    vtx2 = vertex.reshape(E // 128, 128)
    edg2 = edges.reshape(E // 128, 128)
    typ2 = type.reshape(E // 128, 128)
    xe_split = _sc_edges(xlvt_tbl, vtx2, edg2, typ2)
    xe_tbl = xe_split.reshape(2 * NUM_HE, H)
    xv_split = _sc_verts(xe_tbl, vtx2, edg2)
    return _tc_final(xl_split, xv_split, eps2)
